# Initial kernel scaffold; baseline (speedup 1.0000x reference)
import jax
import jax.numpy as jnp
from jax.experimental import pallas as pl


def kernel(x, w, b, gamma, beta, alpha):
    raise NotImplementedError("write your pallas kernel here")


# speedup vs baseline: 1.1624x; 1.1624x over previous
import functools

import jax
import jax.numpy as jnp
from jax.experimental import pallas as pl
from jax.experimental.pallas import tpu as pltpu

_VMEM_LIMIT_BYTES = 48 * 1024 * 1024


def _conv_stats_kernel(xf_ref, wk_ref, conv_ref, stats_ref, *, H, W, Cin, B):
    HW = H * W
    P = W + 1

    col = jax.lax.broadcasted_iota(jnp.int32, (1, HW), 1) % W
    at_left = col == 0
    at_right = col == W - 1
    zpad = jnp.zeros((Cin, P), jnp.bfloat16)

    s_acc = jnp.zeros((wk_ref.shape[0], 1), jnp.float32)
    sq_acc = jnp.zeros((wk_ref.shape[0], 1), jnp.float32)

    for b in range(B):
        xb = xf_ref[b].astype(jnp.bfloat16)
        xp = jnp.concatenate([zpad, xb, zpad], axis=1)
        taps = []
        for ky in range(3):
            for kx in range(3):
                t = xp[:, ky * W + kx:ky * W + kx + HW]
                if kx == 0:
                    t = jnp.where(at_left, jnp.bfloat16(0), t)
                elif kx == 2:
                    t = jnp.where(at_right, jnp.bfloat16(0), t)
                taps.append(t)
        tap_mat = jnp.concatenate(taps, axis=0)
        acc = jnp.dot(wk_ref[...], tap_mat,
                      preferred_element_type=jnp.float32)
        s_acc += jnp.sum(acc, axis=1, keepdims=True)
        sq_acc += jnp.sum(acc * acc, axis=1, keepdims=True)
        conv_ref[b] = acc.astype(jnp.bfloat16)

    stats_ref[0] = jnp.concatenate([s_acc, sq_acc], axis=1)


def _bn_prelu_kernel(conv_ref, scale_ref, shift_ref, alpha_ref, o_ref):
    a = alpha_ref[0, 0]
    for b in range(conv_ref.shape[0]):
        y = conv_ref[b].astype(jnp.float32) * scale_ref[...] + shift_ref[...]
        y = jnp.where(y >= 0.0, y, a * y)
        o_ref[b] = y.astype(o_ref.dtype)


def kernel(x, w, b, gamma, beta, alpha):
    N, Cin, H, W = x.shape
    r = 2
    Cout = Cin * r * r
    HW = H * W
    assert w.shape == (Cout, Cin, 3, 3)
    del b

    xf = x.reshape(N, Cin, HW)

    w3 = jnp.transpose(w, (2, 3, 0, 1)).reshape(9, Cout, Cin)
    wk = jnp.transpose(w3, (1, 0, 2)).reshape(Cout, 9 * Cin).astype(jnp.bfloat16)

    B1 = 2 if N % 2 == 0 else 1
    G1 = N // B1

    conv, stats = pl.pallas_call(
        functools.partial(_conv_stats_kernel, H=H, W=W, Cin=Cin, B=B1),
        out_shape=(jax.ShapeDtypeStruct((N, Cout, HW), jnp.bfloat16),
                   jax.ShapeDtypeStruct((G1, Cout, 2), jnp.float32)),
        grid=(G1,),
        in_specs=[pl.BlockSpec((B1, Cin, HW), lambda i: (i, 0, 0)),
                  pl.BlockSpec((Cout, 9 * Cin), lambda i: (0, 0))],
        out_specs=(pl.BlockSpec((B1, Cout, HW), lambda i: (i, 0, 0)),
                   pl.BlockSpec((1, Cout, 2), lambda i: (i, 0, 0))),
        compiler_params=pltpu.CompilerParams(
            dimension_semantics=("parallel",),
            vmem_limit_bytes=_VMEM_LIMIT_BYTES),
    )(xf, wk)

    count = float(N * HW)
    tot = jnp.sum(stats, axis=0)
    mean = tot[:, 0] / count
    var = tot[:, 1] / count - mean * mean
    inv_std = jax.lax.rsqrt(var + 1e-5)
    scale = (gamma.astype(jnp.float32) * inv_std).reshape(Cout, 1)
    shift = (beta.astype(jnp.float32).reshape(Cout, 1) - mean.reshape(Cout, 1) * scale)
    a2 = jnp.asarray(alpha, jnp.float32).reshape(1, 1)

    B2 = 4 if N % 4 == 0 else 1
    G2 = N // B2

    y = pl.pallas_call(
        _bn_prelu_kernel,
        out_shape=jax.ShapeDtypeStruct((N, Cout, HW), jnp.bfloat16),
        grid=(G2,),
        in_specs=[pl.BlockSpec((B2, Cout, HW), lambda i: (i, 0, 0)),
                  pl.BlockSpec((Cout, 1), lambda i: (0, 0)),
                  pl.BlockSpec((Cout, 1), lambda i: (0, 0)),
                  pl.BlockSpec(memory_space=pltpu.MemorySpace.SMEM)],
        out_specs=pl.BlockSpec((B2, Cout, HW), lambda i: (i, 0, 0)),
        compiler_params=pltpu.CompilerParams(
            dimension_semantics=("parallel",),
            vmem_limit_bytes=_VMEM_LIMIT_BYTES),
    )(conv, scale, shift, a2)

    y = y.reshape(N, Cin, r, r, H, W).astype(x.dtype)
    y = jnp.transpose(y, (0, 1, 4, 2, 5, 3)).reshape(N, Cin, H * r, W * r)
    return y
